# Initial kernel scaffold; baseline (speedup 1.0000x reference)
import jax
import jax.numpy as jnp
from jax.experimental import pallas as pl


def kernel(input_ids, token_type_ids, word_emb, pos_emb, type_emb, gamma, beta):
    raise NotImplementedError("write your pallas kernel here")


# speedup vs baseline: 2.1997x; 2.1997x over previous
import functools

import jax
import jax.numpy as jnp
from jax import lax
from jax.experimental import pallas as pl
from jax.experimental.pallas import tpu as pltpu
from jax.experimental.pallas import tpu_sc as plsc

VOCAB = 30522
HIDDEN = 768
MAX_POS = 2048
B, S = 4, 2048
EPS = 1e-12

NUM_TOKENS = B * S
NC, NS = 2, 16
NW = NC * NS
TOK_PER_W = NUM_TOKENS // NW
CHUNK = 64
NCHUNK = TOK_PER_W // CHUNK

_sc_mesh = plsc.VectorSubcoreMesh(core_axis_name="c", subcore_axis_name="s")


@functools.partial(
    pl.kernel,
    out_type=jax.ShapeDtypeStruct((NUM_TOKENS, HIDDEN), jnp.float32),
    mesh=_sc_mesh,
    scratch_types=[
        pltpu.VMEM((CHUNK,), jnp.int32),
        pltpu.VMEM((CHUNK, HIDDEN), jnp.float32),
        pltpu.SemaphoreType.DMA,
    ],
)
def _sc_gather(table_hbm, idx_hbm, out_hbm, idx_v, rows_v, sem):
    wid = lax.axis_index("s") * NC + lax.axis_index("c")
    base = wid * TOK_PER_W

    @pl.loop(0, NCHUNK)
    def _(c):
        off = base + c * CHUNK
        pltpu.sync_copy(idx_hbm.at[pl.ds(off, CHUNK)], idx_v)
        pltpu.async_copy(table_hbm.at[idx_v], rows_v, sem).wait()
        pltpu.sync_copy(rows_v, out_hbm.at[pl.ds(off, CHUNK)])


ROWS_BLK = 512
GRID = NUM_TOKENS // ROWS_BLK
S_BLKS = S // ROWS_BLK


def _ln_body(words_ref, pos_ref, ttf_ref, type_ref, gamma_ref, beta_ref, out_ref):
    t0 = type_ref[0:1, :]
    tdiff = type_ref[1:2, :] - t0
    x = words_ref[...] + pos_ref[...] + t0 + ttf_ref[...] * tdiff
    mean = jnp.mean(x, axis=-1, keepdims=True)
    xc = x - mean
    var = jnp.mean(xc * xc, axis=-1, keepdims=True)
    normed = xc * lax.rsqrt(var + EPS)
    out_ref[...] = normed * gamma_ref[...] + beta_ref[...]


_ln_call = pl.pallas_call(
    _ln_body,
    grid=(GRID,),
    in_specs=[
        pl.BlockSpec((ROWS_BLK, HIDDEN), lambda i: (i, 0)),
        pl.BlockSpec((ROWS_BLK, HIDDEN), lambda i: (i % S_BLKS, 0)),
        pl.BlockSpec((ROWS_BLK, 1), lambda i: (i, 0)),
        pl.BlockSpec((2, HIDDEN), lambda i: (0, 0)),
        pl.BlockSpec((1, HIDDEN), lambda i: (0, 0)),
        pl.BlockSpec((1, HIDDEN), lambda i: (0, 0)),
    ],
    out_specs=pl.BlockSpec((ROWS_BLK, HIDDEN), lambda i: (i, 0)),
    out_shape=jax.ShapeDtypeStruct((NUM_TOKENS, HIDDEN), jnp.float32),
)


@jax.jit
def kernel(input_ids, token_type_ids, word_emb, pos_emb, type_emb, gamma, beta):
    idx = input_ids.reshape(-1).astype(jnp.int32)
    words = _sc_gather(word_emb, idx)
    ttf = token_type_ids.reshape(NUM_TOKENS, 1).astype(jnp.float32)
    out = _ln_call(
        words,
        pos_emb,
        ttf,
        type_emb,
        gamma.reshape(1, HIDDEN),
        beta.reshape(1, HIDDEN),
    )
    return out.reshape(B, S, HIDDEN)


# speedup vs baseline: 2.3660x; 1.0756x over previous
import functools

import jax
import jax.numpy as jnp
from jax import lax
from jax.experimental import pallas as pl
from jax.experimental.pallas import tpu as pltpu
from jax.experimental.pallas import tpu_sc as plsc

VOCAB = 30522
HIDDEN = 768
MAX_POS = 2048
B, S = 4, 2048
EPS = 1e-12

NUM_TOKENS = B * S
NC, NS = 2, 16
NW = NC * NS
TOK_PER_W = NUM_TOKENS // NW
CHUNK = 64
NCHUNK = TOK_PER_W // CHUNK

_sc_mesh = plsc.VectorSubcoreMesh(core_axis_name="c", subcore_axis_name="s")


@functools.partial(
    pl.kernel,
    out_type=jax.ShapeDtypeStruct((NUM_TOKENS, HIDDEN), jnp.float32),
    mesh=_sc_mesh,
    scratch_types=[
        pltpu.VMEM((TOK_PER_W,), jnp.int32),
        pltpu.VMEM((CHUNK, HIDDEN), jnp.float32),
        pltpu.VMEM((CHUNK, HIDDEN), jnp.float32),
        pltpu.SemaphoreType.DMA,
        pltpu.SemaphoreType.DMA,
        pltpu.SemaphoreType.DMA,
        pltpu.SemaphoreType.DMA,
    ],
)
def _sc_gather(table_hbm, idx_hbm, out_hbm, idx_v, buf0, buf1, gs0, gs1, ws0, ws1):
    wid = lax.axis_index("s") * NC + lax.axis_index("c")
    base = wid * TOK_PER_W
    pltpu.sync_copy(idx_hbm.at[pl.ds(base, TOK_PER_W)], idx_v)

    bufs = (buf0, buf1)
    gsems = (gs0, gs1)
    wsems = (ws0, ws1)

    def gather_start(c):
        return pltpu.async_copy(
            table_hbm.at[idx_v.at[pl.ds(c * CHUNK, CHUNK)]], bufs[c % 2],
            gsems[c % 2])

    def write_start(c):
        return pltpu.async_copy(
            bufs[c % 2], out_hbm.at[pl.ds(base + c * CHUNK, CHUNK)],
            wsems[c % 2])

    g = [gather_start(0), gather_start(1)]
    w = []
    for c in range(NCHUNK):
        g[c].wait()
        w.append(write_start(c))
        if c + 2 < NCHUNK:
            w[c].wait()
            g.append(gather_start(c + 2))
    w[-2].wait()
    w[-1].wait()


ROWS_BLK = 512
S_BLKS = S // ROWS_BLK


def _ln_body(words_ref, pos_ref, tt_ref, type_ref, gamma_ref, beta_ref, out_ref):
    t0 = type_ref[0:1, :]
    tdiff = type_ref[1:2, :] - t0
    ttf = tt_ref[...].astype(jnp.float32)
    x = words_ref[...] + pos_ref[...] + t0 + ttf * tdiff
    mean = jnp.mean(x, axis=-1, keepdims=True)
    xc = x - mean
    var = jnp.mean(xc * xc, axis=-1, keepdims=True)
    normed = xc * lax.rsqrt(var + EPS)
    out_ref[...] = normed * gamma_ref[...] + beta_ref[...]


_ln_call = pl.pallas_call(
    _ln_body,
    grid=(S_BLKS, B),
    in_specs=[
        pl.BlockSpec((ROWS_BLK, HIDDEN), lambda i, j: (j * S_BLKS + i, 0)),
        pl.BlockSpec((ROWS_BLK, HIDDEN), lambda i, j: (i, 0)),
        pl.BlockSpec((ROWS_BLK, 1), lambda i, j: (j * S_BLKS + i, 0)),
        pl.BlockSpec((2, HIDDEN), lambda i, j: (0, 0)),
        pl.BlockSpec((1, HIDDEN), lambda i, j: (0, 0)),
        pl.BlockSpec((1, HIDDEN), lambda i, j: (0, 0)),
    ],
    out_specs=pl.BlockSpec((ROWS_BLK, HIDDEN), lambda i, j: (j * S_BLKS + i, 0)),
    out_shape=jax.ShapeDtypeStruct((NUM_TOKENS, HIDDEN), jnp.float32),
)


@jax.jit
def kernel(input_ids, token_type_ids, word_emb, pos_emb, type_emb, gamma, beta):
    idx = input_ids.reshape(-1).astype(jnp.int32)
    words = _sc_gather(word_emb, idx)
    tt = token_type_ids.reshape(NUM_TOKENS, 1).astype(jnp.int32)
    out = _ln_call(
        words,
        pos_emb,
        tt,
        type_emb,
        gamma.reshape(1, HIDDEN),
        beta.reshape(1, HIDDEN),
    )
    return out.reshape(B, S, HIDDEN)
